# baseline (speedup 1.0000x reference)
import functools

import jax
import jax.numpy as jnp
from jax import lax
from jax.experimental import pallas as pl
from jax.experimental.pallas import tpu as pltpu
from jax.experimental.pallas import tpu_sc as plsc

_B = 4096
_LG = 200
_LT = 400
_V = 1000
_D = 32
_H = 256
_OUT = 64
_W = 1024
_TSCALE = 4096.0

_NC = 2
_NS = 16
_LANES = 16
_NW = _NC * _NS
_RPW = _B // _NW
_R = 16
_NCHUNK = _RPW // _R
_GRP = 8


def _sc_histogram(gid, tid, nrows):
    mesh = plsc.VectorSubcoreMesh(core_axis_name="c", subcore_axis_name="s")
    rpw = nrows // _NW
    nchunk = rpw // _R

    @functools.partial(
        pl.kernel,
        mesh=mesh,
        compiler_params=pltpu.CompilerParams(needs_layout_passes=False),
        out_type=jax.ShapeDtypeStruct((8 * nrows * 128,), jnp.float32),
        scratch_types=[
            pltpu.VMEM((_R, _LG), jnp.int32),
            pltpu.VMEM((_R, _LG), jnp.int32),
            pltpu.VMEM((_R, _LT), jnp.int32),
            pltpu.VMEM((_R, _LT), jnp.int32),
            pltpu.VMEM((_R * _W,), jnp.float32),
            pltpu.VMEM((_R * _W,), jnp.float32),
            pltpu.SemaphoreType.DMA,
            pltpu.SemaphoreType.DMA,
            pltpu.SemaphoreType.DMA,
            pltpu.SemaphoreType.DMA,
            pltpu.SemaphoreType.DMA,
            pltpu.SemaphoreType.DMA,
        ],
    )
    def hist_kernel(gid_hbm, tid_hbm, out_hbm, gid_v0, gid_v1, tid_v0, tid_v1,
                    hist_v0, hist_v1, sg0, sg1, st0, st1, so0, so1):
        gids = (gid_v0, gid_v1)
        tids = (tid_v0, tid_v1)
        hists = (hist_v0, hist_v1)
        sgs = (sg0, sg1)
        sts = (st0, st1)
        sos = (so0, so1)
        wid = lax.axis_index("s") * _NC + lax.axis_index("c")
        base = wid * rpw
        ones = jnp.ones((_LANES,), jnp.float32)
        tval = jnp.full((_LANES,), _TSCALE, jnp.float32)
        lane = lax.iota(jnp.int32, _LANES)
        tailmask = lane >= (16 - _LG % 16)
        zero16 = jnp.zeros((16,), jnp.float32)

        def start_id_load(c, b):
            row0 = base + c * _R
            pltpu.async_copy(gid_hbm.at[pl.ds(row0, _R)], gids[b], sgs[b])
            pltpu.async_copy(tid_hbm.at[pl.ds(row0, _R)], tids[b], sts[b])

        def wait_id(b):
            pltpu.make_async_copy(
                gid_hbm.at[pl.ds(0, _R)], gids[b], sgs[b]).wait()
            pltpu.make_async_copy(
                tid_hbm.at[pl.ds(0, _R)], tids[b], sts[b]).wait()

        start_id_load(0, 0)
        start_id_load(1, 1)

        def outer_body(o, carry):
            for b in range(2):
                c = o * 2 + b
                row0 = base + c * _R
                hist_b = hists[b]
                gid_b = gids[b]
                tid_b = tids[b]

                @pl.when(c >= 2)
                def _wait_out():
                    for s in range(8):
                        pltpu.make_async_copy(
                            hist_b.at[pl.ds(0, _R * 128)],
                            out_hbm.at[pl.ds(0, _R * 128)], sos[b]).wait()

                def zero_body(j, carry2):
                    for k in range(16):
                        hist_b[pl.ds(j * 256 + k * 16, 16)] = zero16
                    return carry2

                lax.fori_loop(0, (_R * _W) // 256, zero_body, 0)

                wait_id(b)

                def row_body(r, carry2):
                    rb = jnp.full((_LANES,), r * 128, jnp.int32)
                    chunks = (
                        [(cc * 16, gid_b, ones, None)
                         for cc in range(_LG // 16)]
                        + [(_LG - 16, gid_b, ones, tailmask)]
                        + [(cc * 16, tid_b, tval, None)
                           for cc in range(_LT // 16)]
                    )
                    for i in range(0, len(chunks), _GRP):
                        grp = chunks[i:i + _GRP]
                        vals = [ref[r, pl.ds(off, 16)]
                                for off, ref, _, _ in grp]
                        idxs = [((v & 896) << 4) + (v & 127) + rb
                                for v in vals]
                        for idx, (_, _, val, msk) in zip(idxs, grp):
                            if msk is None:
                                plsc.addupdate_scatter(hist_b, [idx], val)
                            else:
                                plsc.addupdate_scatter(
                                    hist_b, [idx], val, mask=msk)
                    return carry2

                lax.fori_loop(0, _R, row_body, 0)

                @pl.when(c + 2 < nchunk)
                def _prefetch():
                    start_id_load(c + 2, b)

                for s in range(8):
                    pltpu.async_copy(
                        hist_b.at[pl.ds(s * _R * 128, _R * 128)],
                        out_hbm.at[pl.ds((s * nrows + row0) * 128, _R * 128)],
                        sos[b])
            return carry

        lax.fori_loop(0, nchunk // 2, outer_body, 0)

        for b in range(2):
            for s in range(8):
                pltpu.make_async_copy(
                    hists[b].at[pl.ds(0, _R * 128)],
                    out_hbm.at[pl.ds(0, _R * 128)], sos[b]).wait()

    return hist_kernel(gid, tid)


def _tc_towers(counts2, Tg2, Tt2, Wg1, bg1, Wg2, bg2, Wg3, bg3,
               Wt1, bt1, Wt2, bt2, Wt3, bt3):
    nrows = counts2.shape[1]
    BB = min(2048, nrows)
    NCT = _W // 128

    def body(c_ref, tg_ref, tt_ref, wg1, bg1r, wg2, bg2r, wg3, bg3r,
             wt1, bt1r, wt2, bt2r, wt3, bt3r, out_ref, x_s):
        ct = pl.program_id(1)
        v = c_ref[0]
        tc_cnt = jnp.floor(v * (1.0 / 4096.0))
        gc_cnt = v - tc_cnt * 4096.0
        p = (jnp.dot(gc_cnt.astype(jnp.bfloat16), tg_ref[...],
                     preferred_element_type=jnp.float32)
             + jnp.dot(tc_cnt.astype(jnp.bfloat16), tt_ref[...],
                       preferred_element_type=jnp.float32))

        @pl.when(ct == 0)
        def _init():
            x_s[...] = p

        @pl.when(ct > 0)
        def _acc():
            x_s[...] = x_s[...] + p

        @pl.when(ct == NCT - 1)
        def _finish():
            x = x_s[...]
            g = (x[:, 0:_D]
                 / jnp.maximum(x[:, _D:_D + 1], 1.0)).astype(jnp.bfloat16)
            t = (x[:, 64:64 + _D]
                 / jnp.maximum(x[:, 64 + _D:64 + _D + 1],
                               1.0)).astype(jnp.bfloat16)
            hg = jnp.maximum(
                jnp.dot(g, wg1[...], preferred_element_type=jnp.float32)
                + bg1r[...], 0.0).astype(jnp.bfloat16)
            hg = jnp.maximum(
                jnp.dot(hg, wg2[...], preferred_element_type=jnp.float32)
                + bg2r[...], 0.0).astype(jnp.bfloat16)
            gv = (jnp.dot(hg, wg3[...], preferred_element_type=jnp.float32)
                  + bg3r[...])
            ht = jnp.maximum(
                jnp.dot(t, wt1[...], preferred_element_type=jnp.float32)
                + bt1r[...], 0.0).astype(jnp.bfloat16)
            ht = jnp.maximum(
                jnp.dot(ht, wt2[...], preferred_element_type=jnp.float32)
                + bt2r[...], 0.0).astype(jnp.bfloat16)
            tv = (jnp.dot(ht, wt3[...], preferred_element_type=jnp.float32)
                  + bt3r[...])
            out_ref[...] = jax.nn.sigmoid(jnp.sum(gv * tv, axis=1))

    full = lambda shape: pl.BlockSpec(shape, lambda i, ct: (0,) * len(shape))
    return pl.pallas_call(
        body,
        grid=(nrows // BB, NCT),
        in_specs=[
            pl.BlockSpec((1, BB, 128), lambda i, ct: (ct, i, 0)),
            pl.BlockSpec((128, 128), lambda i, ct: (ct, 0)),
            pl.BlockSpec((128, 128), lambda i, ct: (ct, 0)),
            full((_D, _H)), full((1, _H)),
            full((_H, _H)), full((1, _H)),
            full((_H, _OUT)), full((1, _OUT)),
            full((_D, _H)), full((1, _H)),
            full((_H, _H)), full((1, _H)),
            full((_H, _OUT)), full((1, _OUT)),
        ],
        out_specs=pl.BlockSpec((BB,), lambda i, ct: (i,)),
        out_shape=jax.ShapeDtypeStruct((nrows,), jnp.float32),
        scratch_shapes=[pltpu.VMEM((BB, 128), jnp.float32)],
    )(counts2, Tg2.astype(jnp.bfloat16), Tt2.astype(jnp.bfloat16),
      Wg1.astype(jnp.bfloat16), bg1.reshape(1, _H),
      Wg2.astype(jnp.bfloat16), bg2.reshape(1, _H),
      Wg3.astype(jnp.bfloat16), bg3.reshape(1, _OUT),
      Wt1.astype(jnp.bfloat16), bt1.reshape(1, _H),
      Wt2.astype(jnp.bfloat16), bt2.reshape(1, _H),
      Wt3.astype(jnp.bfloat16), bt3.reshape(1, _OUT))


def kernel(group_ids, tech_ids, Eg, Et, Wg1, bg1, Wg2, bg2, Wg3, bg3,
           Wt1, bt1, Wt2, bt2, Wt3, bt3):
    splits = (1024, 1024, 2048)
    offs = [0]
    for s in splits:
        offs.append(offs[-1] + s)
    counts_q = [
        _sc_histogram(group_ids[o:o + s], tech_ids[o:o + s], s)
        for o, s in zip(offs, splits)
    ]

    m = (jnp.arange(_V) != 0).astype(jnp.float32)[:, None]
    z = jnp.zeros((_V, 1), jnp.float32)
    tg = jnp.concatenate(
        [Eg * m, m] + [z] * (128 - _D - 1), axis=1)
    tt = jnp.concatenate(
        [z] * 64 + [Et * m, m] + [z] * (128 - 64 - _D - 1), axis=1)
    Tg = jnp.pad(tg, ((0, _W - _V), (0, 0)))
    Tt = jnp.pad(tt, ((0, _W - _V), (0, 0)))

    outs = [
        _tc_towers(c.reshape(8, c.shape[0] // (8 * 128), 128), Tg, Tt,
                   Wg1, bg1, Wg2, bg2, Wg3, bg3,
                   Wt1, bt1, Wt2, bt2, Wt3, bt3)
        for c in counts_q
    ]
    return jnp.concatenate(outs, axis=0)


# speedup vs baseline: 1.0139x; 1.0139x over previous
import functools

import jax
import jax.numpy as jnp
from jax import lax
from jax.experimental import pallas as pl
from jax.experimental.pallas import tpu as pltpu
from jax.experimental.pallas import tpu_sc as plsc

_B = 4096
_LG = 200
_LT = 400
_V = 1000
_D = 32
_H = 256
_OUT = 64
_W = 1024
_TSCALE = 4096.0

_NC = 2
_NS = 16
_LANES = 16
_NW = _NC * _NS
_RPW = _B // _NW
_R = 16
_NCHUNK = _RPW // _R
_GRP = 8


def _sc_histogram(gid, tid, nrows):
    mesh = plsc.VectorSubcoreMesh(core_axis_name="c", subcore_axis_name="s")
    rpw = nrows // _NW
    nchunk = rpw // _R

    @functools.partial(
        pl.kernel,
        mesh=mesh,
        compiler_params=pltpu.CompilerParams(needs_layout_passes=False),
        out_type=jax.ShapeDtypeStruct((8 * nrows * 128,), jnp.float32),
        scratch_types=[
            pltpu.VMEM((_R, _LG), jnp.int16),
            pltpu.VMEM((_R, _LG), jnp.int16),
            pltpu.VMEM((_R, _LT), jnp.int16),
            pltpu.VMEM((_R, _LT), jnp.int16),
            pltpu.VMEM((_R * _W,), jnp.float32),
            pltpu.VMEM((_R * _W,), jnp.float32),
            pltpu.SemaphoreType.DMA,
            pltpu.SemaphoreType.DMA,
            pltpu.SemaphoreType.DMA,
            pltpu.SemaphoreType.DMA,
            pltpu.SemaphoreType.DMA,
            pltpu.SemaphoreType.DMA,
        ],
    )
    def hist_kernel(gid_hbm, tid_hbm, out_hbm, gid_v0, gid_v1, tid_v0, tid_v1,
                    hist_v0, hist_v1, sg0, sg1, st0, st1, so0, so1):
        gids = (gid_v0, gid_v1)
        tids = (tid_v0, tid_v1)
        hists = (hist_v0, hist_v1)
        sgs = (sg0, sg1)
        sts = (st0, st1)
        sos = (so0, so1)
        wid = lax.axis_index("s") * _NC + lax.axis_index("c")
        base = wid * rpw
        ones = jnp.ones((_LANES,), jnp.float32)
        tval = jnp.full((_LANES,), _TSCALE, jnp.float32)
        lane = lax.iota(jnp.int32, _LANES)
        gtailmask = lane >= 8
        never = lane < 0
        always = lane >= 0
        zero16 = jnp.zeros((16,), jnp.float32)

        def start_id_load(c, b):
            row0 = base + c * _R
            pltpu.async_copy(gid_hbm.at[pl.ds(row0, _R)], gids[b], sgs[b])
            pltpu.async_copy(tid_hbm.at[pl.ds(row0, _R)], tids[b], sts[b])

        def wait_id(b):
            pltpu.make_async_copy(
                gid_hbm.at[pl.ds(0, _R)], gids[b], sgs[b]).wait()
            pltpu.make_async_copy(
                tid_hbm.at[pl.ds(0, _R)], tids[b], sts[b]).wait()

        start_id_load(0, 0)
        start_id_load(1, 1)

        def outer_body(o, carry):
            for b in range(2):
                c = o * 2 + b
                row0 = base + c * _R
                hist_b = hists[b]
                gid_b = gids[b]
                tid_b = tids[b]

                @pl.when(c >= 2)
                def _wait_out():
                    for s in range(8):
                        pltpu.make_async_copy(
                            hist_b.at[pl.ds(0, _R * 128)],
                            out_hbm.at[pl.ds(0, _R * 128)], sos[b]).wait()

                def zero_body(j, carry2):
                    for k in range(16):
                        hist_b[pl.ds(j * 256 + k * 16, 16)] = zero16
                    return carry2

                lax.fori_loop(0, (_R * _W) // 256, zero_body, 0)

                wait_id(b)

                def row_body(r, carry2):
                    rb = jnp.full((_LANES,), r * 128, jnp.int32)
                    chunks = (
                        [(cc * 32, gid_b, ones, None, None)
                         for cc in range(_LG // 32)]
                        + [(168, gid_b, ones, never, gtailmask)]
                        + [(cc * 32, tid_b, tval, None, None)
                           for cc in range(_LT // 32)]
                        + [(368, tid_b, tval, never, always)]
                    )
                    for i in range(0, len(chunks), 4):
                        grp = chunks[i:i + 4]
                        vals = [plsc.unpack(
                                    ref[r, pl.ds(off, 32)],
                                    format=plsc.PackFormat.INTERLEAVED,
                                    preferred_element_type=jnp.int32)
                                for off, ref, _, _, _ in grp]
                        idxs = [((v & 896) << 4) + (v & 127) + rb
                                for ab in vals for v in ab]
                        flat = [(val, msk)
                                for _, _, val, ma, mb in grp
                                for msk in (ma, mb)]
                        for idx, (val, msk) in zip(idxs, flat):
                            if msk is None:
                                plsc.addupdate_scatter(hist_b, [idx], val)
                            else:
                                plsc.addupdate_scatter(
                                    hist_b, [idx], val, mask=msk)
                    return carry2

                lax.fori_loop(0, _R, row_body, 0)

                @pl.when(c + 2 < nchunk)
                def _prefetch():
                    start_id_load(c + 2, b)

                for s in range(8):
                    pltpu.async_copy(
                        hist_b.at[pl.ds(s * _R * 128, _R * 128)],
                        out_hbm.at[pl.ds((s * nrows + row0) * 128, _R * 128)],
                        sos[b])
            return carry

        lax.fori_loop(0, nchunk // 2, outer_body, 0)

        for b in range(2):
            for s in range(8):
                pltpu.make_async_copy(
                    hists[b].at[pl.ds(0, _R * 128)],
                    out_hbm.at[pl.ds(0, _R * 128)], sos[b]).wait()

    return hist_kernel(gid, tid)


def _tc_towers(counts2, Tg2, Tt2, Wg1, bg1, Wg2, bg2, Wg3, bg3,
               Wt1, bt1, Wt2, bt2, Wt3, bt3):
    nrows = counts2.shape[1]
    BB = min(2048, nrows)
    NCT = _W // 128

    def body(c_ref, tg_ref, tt_ref, wg1, bg1r, wg2, bg2r, wg3, bg3r,
             wt1, bt1r, wt2, bt2r, wt3, bt3r, out_ref, x_s):
        ct = pl.program_id(1)
        v = c_ref[0]
        tc_cnt = jnp.floor(v * (1.0 / 4096.0))
        gc_cnt = v - tc_cnt * 4096.0
        p = (jnp.dot(gc_cnt.astype(jnp.bfloat16), tg_ref[...],
                     preferred_element_type=jnp.float32)
             + jnp.dot(tc_cnt.astype(jnp.bfloat16), tt_ref[...],
                       preferred_element_type=jnp.float32))

        @pl.when(ct == 0)
        def _init():
            x_s[...] = p

        @pl.when(ct > 0)
        def _acc():
            x_s[...] = x_s[...] + p

        @pl.when(ct == NCT - 1)
        def _finish():
            x = x_s[...]
            g = (x[:, 0:_D]
                 / jnp.maximum(x[:, _D:_D + 1], 1.0)).astype(jnp.bfloat16)
            t = (x[:, 64:64 + _D]
                 / jnp.maximum(x[:, 64 + _D:64 + _D + 1],
                               1.0)).astype(jnp.bfloat16)
            hg = jnp.maximum(
                jnp.dot(g, wg1[...], preferred_element_type=jnp.float32)
                + bg1r[...], 0.0).astype(jnp.bfloat16)
            hg = jnp.maximum(
                jnp.dot(hg, wg2[...], preferred_element_type=jnp.float32)
                + bg2r[...], 0.0).astype(jnp.bfloat16)
            gv = (jnp.dot(hg, wg3[...], preferred_element_type=jnp.float32)
                  + bg3r[...])
            ht = jnp.maximum(
                jnp.dot(t, wt1[...], preferred_element_type=jnp.float32)
                + bt1r[...], 0.0).astype(jnp.bfloat16)
            ht = jnp.maximum(
                jnp.dot(ht, wt2[...], preferred_element_type=jnp.float32)
                + bt2r[...], 0.0).astype(jnp.bfloat16)
            tv = (jnp.dot(ht, wt3[...], preferred_element_type=jnp.float32)
                  + bt3r[...])
            out_ref[...] = jax.nn.sigmoid(jnp.sum(gv * tv, axis=1))

    full = lambda shape: pl.BlockSpec(shape, lambda i, ct: (0,) * len(shape))
    return pl.pallas_call(
        body,
        grid=(nrows // BB, NCT),
        in_specs=[
            pl.BlockSpec((1, BB, 128), lambda i, ct: (ct, i, 0)),
            pl.BlockSpec((128, 128), lambda i, ct: (ct, 0)),
            pl.BlockSpec((128, 128), lambda i, ct: (ct, 0)),
            full((_D, _H)), full((1, _H)),
            full((_H, _H)), full((1, _H)),
            full((_H, _OUT)), full((1, _OUT)),
            full((_D, _H)), full((1, _H)),
            full((_H, _H)), full((1, _H)),
            full((_H, _OUT)), full((1, _OUT)),
        ],
        out_specs=pl.BlockSpec((BB,), lambda i, ct: (i,)),
        out_shape=jax.ShapeDtypeStruct((nrows,), jnp.float32),
        scratch_shapes=[pltpu.VMEM((BB, 128), jnp.float32)],
    )(counts2, Tg2.astype(jnp.bfloat16), Tt2.astype(jnp.bfloat16),
      Wg1.astype(jnp.bfloat16), bg1.reshape(1, _H),
      Wg2.astype(jnp.bfloat16), bg2.reshape(1, _H),
      Wg3.astype(jnp.bfloat16), bg3.reshape(1, _OUT),
      Wt1.astype(jnp.bfloat16), bt1.reshape(1, _H),
      Wt2.astype(jnp.bfloat16), bt2.reshape(1, _H),
      Wt3.astype(jnp.bfloat16), bt3.reshape(1, _OUT))


def kernel(group_ids, tech_ids, Eg, Et, Wg1, bg1, Wg2, bg2, Wg3, bg3,
           Wt1, bt1, Wt2, bt2, Wt3, bt3):
    nsplit = 2
    qr = _B // nsplit
    gid16 = group_ids.astype(jnp.int16)
    tid16 = tech_ids.astype(jnp.int16)
    counts_q = [
        _sc_histogram(gid16[i * qr:(i + 1) * qr],
                      tid16[i * qr:(i + 1) * qr], qr)
        for i in range(nsplit)
    ]

    m = (jnp.arange(_V) != 0).astype(jnp.float32)[:, None]
    z = jnp.zeros((_V, 1), jnp.float32)
    tg = jnp.concatenate(
        [Eg * m, m] + [z] * (128 - _D - 1), axis=1)
    tt = jnp.concatenate(
        [z] * 64 + [Et * m, m] + [z] * (128 - 64 - _D - 1), axis=1)
    Tg = jnp.pad(tg, ((0, _W - _V), (0, 0)))
    Tt = jnp.pad(tt, ((0, _W - _V), (0, 0)))

    outs = [
        _tc_towers(c.reshape(8, c.shape[0] // (8 * 128), 128), Tg, Tt,
                   Wg1, bg1, Wg2, bg2, Wg3, bg3,
                   Wt1, bt1, Wt2, bt2, Wt3, bt3)
        for c in counts_q
    ]
    return jnp.concatenate(outs, axis=0)


# speedup vs baseline: 1.0318x; 1.0177x over previous
import functools

import jax
import jax.numpy as jnp
from jax import lax
from jax.experimental import pallas as pl
from jax.experimental.pallas import tpu as pltpu
from jax.experimental.pallas import tpu_sc as plsc

_B = 4096
_LG = 200
_LT = 400
_V = 1000
_D = 32
_H = 256
_OUT = 64
_W = 1024
_TSCALE = 4096.0

_NC = 2
_NS = 16
_LANES = 16
_NW = _NC * _NS
_RPW = _B // _NW
_R = 16
_NCHUNK = _RPW // _R
_GRP = 8


def _sc_histogram(gid, tid, nrows):
    mesh = plsc.VectorSubcoreMesh(core_axis_name="c", subcore_axis_name="s")
    rpw = nrows // _NW
    nchunk = rpw // _R

    @functools.partial(
        pl.kernel,
        mesh=mesh,
        compiler_params=pltpu.CompilerParams(needs_layout_passes=False),
        out_type=jax.ShapeDtypeStruct((8 * nrows * 128,), jnp.float32),
        scratch_types=[
            pltpu.VMEM((_R, _LG), jnp.int32),
            pltpu.VMEM((_R, _LG), jnp.int32),
            pltpu.VMEM((_R, _LT), jnp.int32),
            pltpu.VMEM((_R, _LT), jnp.int32),
            pltpu.VMEM((_R * _W,), jnp.float32),
            pltpu.VMEM((_R * _W,), jnp.float32),
            pltpu.SemaphoreType.DMA,
            pltpu.SemaphoreType.DMA,
            pltpu.SemaphoreType.DMA,
            pltpu.SemaphoreType.DMA,
            pltpu.SemaphoreType.DMA,
            pltpu.SemaphoreType.DMA,
        ],
    )
    def hist_kernel(gid_hbm, tid_hbm, out_hbm, gid_v0, gid_v1, tid_v0, tid_v1,
                    hist_v0, hist_v1, sg0, sg1, st0, st1, so0, so1):
        gids = (gid_v0, gid_v1)
        tids = (tid_v0, tid_v1)
        hists = (hist_v0, hist_v1)
        sgs = (sg0, sg1)
        sts = (st0, st1)
        sos = (so0, so1)
        wid = lax.axis_index("s") * _NC + lax.axis_index("c")
        base = wid * rpw
        ones = jnp.ones((_LANES,), jnp.float32)
        tval = jnp.full((_LANES,), _TSCALE, jnp.float32)
        lane = lax.iota(jnp.int32, _LANES)
        tailmask = lane >= (16 - _LG % 16)
        zero16 = jnp.zeros((16,), jnp.float32)

        def start_id_load(c, b):
            row0 = base + c * _R
            pltpu.async_copy(gid_hbm.at[pl.ds(row0, _R)], gids[b], sgs[b])
            pltpu.async_copy(tid_hbm.at[pl.ds(row0, _R)], tids[b], sts[b])

        def wait_id(b):
            pltpu.make_async_copy(
                gid_hbm.at[pl.ds(0, _R)], gids[b], sgs[b]).wait()
            pltpu.make_async_copy(
                tid_hbm.at[pl.ds(0, _R)], tids[b], sts[b]).wait()

        start_id_load(0, 0)
        start_id_load(1, 1)

        def outer_body(o, carry):
            for b in range(2):
                c = o * 2 + b
                row0 = base + c * _R
                hist_b = hists[b]
                gid_b = gids[b]
                tid_b = tids[b]

                @pl.when(c >= 2)
                def _wait_out():
                    for s in range(8):
                        pltpu.make_async_copy(
                            hist_b.at[pl.ds(0, _R * 128)],
                            out_hbm.at[pl.ds(0, _R * 128)], sos[b]).wait()

                def zero_body(j, carry2):
                    for k in range(16):
                        hist_b[pl.ds(j * 256 + k * 16, 16)] = zero16
                    return carry2

                lax.fori_loop(0, (_R * _W) // 256, zero_body, 0)

                wait_id(b)

                def row_body(r, carry2):
                    rb = jnp.full((_LANES,), r * 128, jnp.int32)
                    chunks = (
                        [(cc * 16, gid_b, ones, None)
                         for cc in range(_LG // 16)]
                        + [(_LG - 16, gid_b, ones, tailmask)]
                        + [(cc * 16, tid_b, tval, None)
                           for cc in range(_LT // 16)]
                    )
                    for i in range(0, len(chunks), _GRP):
                        grp = chunks[i:i + _GRP]
                        vals = [ref[r, pl.ds(off, 16)]
                                for off, ref, _, _ in grp]
                        idxs = [((v & 896) << 4) + (v & 127) + rb
                                for v in vals]
                        for idx, (_, _, val, msk) in zip(idxs, grp):
                            if msk is None:
                                plsc.addupdate_scatter(hist_b, [idx], val)
                            else:
                                plsc.addupdate_scatter(
                                    hist_b, [idx], val, mask=msk)
                    return carry2

                lax.fori_loop(0, _R, row_body, 0)

                @pl.when(c + 2 < nchunk)
                def _prefetch():
                    start_id_load(c + 2, b)

                for s in range(8):
                    pltpu.async_copy(
                        hist_b.at[pl.ds(s * _R * 128, _R * 128)],
                        out_hbm.at[pl.ds((s * nrows + row0) * 128, _R * 128)],
                        sos[b])
            return carry

        lax.fori_loop(0, nchunk // 2, outer_body, 0)

        for b in range(2):
            for s in range(8):
                pltpu.make_async_copy(
                    hists[b].at[pl.ds(0, _R * 128)],
                    out_hbm.at[pl.ds(0, _R * 128)], sos[b]).wait()

    return hist_kernel(gid, tid)


def _tc_towers(counts2, Tg2, Tt2, Wg1, bg1, Wg2, bg2, Wg3, bg3,
               Wt1, bt1, Wt2, bt2, Wt3, bt3):
    nrows = counts2.shape[1]
    BB = min(2048, nrows)
    NCT = _W // 128

    def body(c_ref, tg_ref, tt_ref, wg1, bg1r, wg2, bg2r, wg3, bg3r,
             wt1, bt1r, wt2, bt2r, wt3, bt3r, out_ref, x_s):
        ct = pl.program_id(1)
        v = c_ref[0]
        tc_cnt = jnp.floor(v * (1.0 / 4096.0))
        gc_cnt = v - tc_cnt * 4096.0
        p = (jnp.dot(gc_cnt.astype(jnp.bfloat16), tg_ref[...],
                     preferred_element_type=jnp.float32)
             + jnp.dot(tc_cnt.astype(jnp.bfloat16), tt_ref[...],
                       preferred_element_type=jnp.float32))

        @pl.when(ct == 0)
        def _init():
            x_s[...] = p

        @pl.when(ct > 0)
        def _acc():
            x_s[...] = x_s[...] + p

        @pl.when(ct == NCT - 1)
        def _finish():
            x = x_s[...]
            g = (x[:, 0:_D]
                 / jnp.maximum(x[:, _D:_D + 1], 1.0)).astype(jnp.bfloat16)
            t = (x[:, 64:64 + _D]
                 / jnp.maximum(x[:, 64 + _D:64 + _D + 1],
                               1.0)).astype(jnp.bfloat16)
            hg = jnp.maximum(
                jnp.dot(g, wg1[...], preferred_element_type=jnp.float32)
                + bg1r[...], 0.0).astype(jnp.bfloat16)
            hg = jnp.maximum(
                jnp.dot(hg, wg2[...], preferred_element_type=jnp.float32)
                + bg2r[...], 0.0).astype(jnp.bfloat16)
            gv = (jnp.dot(hg, wg3[...], preferred_element_type=jnp.float32)
                  + bg3r[...])
            ht = jnp.maximum(
                jnp.dot(t, wt1[...], preferred_element_type=jnp.float32)
                + bt1r[...], 0.0).astype(jnp.bfloat16)
            ht = jnp.maximum(
                jnp.dot(ht, wt2[...], preferred_element_type=jnp.float32)
                + bt2r[...], 0.0).astype(jnp.bfloat16)
            tv = (jnp.dot(ht, wt3[...], preferred_element_type=jnp.float32)
                  + bt3r[...])
            out_ref[...] = jax.nn.sigmoid(jnp.sum(gv * tv, axis=1))

    full = lambda shape: pl.BlockSpec(shape, lambda i, ct: (0,) * len(shape))
    return pl.pallas_call(
        body,
        grid=(nrows // BB, NCT),
        in_specs=[
            pl.BlockSpec((1, BB, 128), lambda i, ct: (ct, i, 0)),
            pl.BlockSpec((128, 128), lambda i, ct: (ct, 0)),
            pl.BlockSpec((128, 128), lambda i, ct: (ct, 0)),
            full((_D, _H)), full((1, _H)),
            full((_H, _H)), full((1, _H)),
            full((_H, _OUT)), full((1, _OUT)),
            full((_D, _H)), full((1, _H)),
            full((_H, _H)), full((1, _H)),
            full((_H, _OUT)), full((1, _OUT)),
        ],
        out_specs=pl.BlockSpec((BB,), lambda i, ct: (i,)),
        out_shape=jax.ShapeDtypeStruct((nrows,), jnp.float32),
        scratch_shapes=[pltpu.VMEM((BB, 128), jnp.float32)],
    )(counts2, Tg2.astype(jnp.bfloat16), Tt2.astype(jnp.bfloat16),
      Wg1.astype(jnp.bfloat16), bg1.reshape(1, _H),
      Wg2.astype(jnp.bfloat16), bg2.reshape(1, _H),
      Wg3.astype(jnp.bfloat16), bg3.reshape(1, _OUT),
      Wt1.astype(jnp.bfloat16), bt1.reshape(1, _H),
      Wt2.astype(jnp.bfloat16), bt2.reshape(1, _H),
      Wt3.astype(jnp.bfloat16), bt3.reshape(1, _OUT))


def kernel(group_ids, tech_ids, Eg, Et, Wg1, bg1, Wg2, bg2, Wg3, bg3,
           Wt1, bt1, Wt2, bt2, Wt3, bt3):
    nsplit = 2
    qr = _B // nsplit
    counts_q = [
        _sc_histogram(group_ids[i * qr:(i + 1) * qr],
                      tech_ids[i * qr:(i + 1) * qr], qr)
        for i in range(nsplit)
    ]

    m = (jnp.arange(_V) != 0).astype(jnp.float32)[:, None]
    z = jnp.zeros((_V, 1), jnp.float32)
    tg = jnp.concatenate(
        [Eg * m, m] + [z] * (128 - _D - 1), axis=1)
    tt = jnp.concatenate(
        [z] * 64 + [Et * m, m] + [z] * (128 - 64 - _D - 1), axis=1)
    Tg = jnp.pad(tg, ((0, _W - _V), (0, 0)))
    Tt = jnp.pad(tt, ((0, _W - _V), (0, 0)))

    outs = [
        _tc_towers(c.reshape(8, c.shape[0] // (8 * 128), 128), Tg, Tt,
                   Wg1, bg1, Wg2, bg2, Wg3, bg3,
                   Wt1, bt1, Wt2, bt2, Wt3, bt3)
        for c in counts_q
    ]
    return jnp.concatenate(outs, axis=0)


# speedup vs baseline: 1.0324x; 1.0005x over previous
import functools

import jax
import jax.numpy as jnp
from jax import lax
from jax.experimental import pallas as pl
from jax.experimental.pallas import tpu as pltpu
from jax.experimental.pallas import tpu_sc as plsc

_B = 4096
_LG = 200
_LT = 400
_V = 1000
_D = 32
_H = 256
_OUT = 64
_W = 1024
_TSCALE = 4096.0

_NC = 2
_NS = 16
_LANES = 16
_NW = _NC * _NS
_RPW = _B // _NW
_R = 16
_NCHUNK = _RPW // _R
_GRP = 8


def _sc_histogram(gid, tid, nrows):
    mesh = plsc.VectorSubcoreMesh(core_axis_name="c", subcore_axis_name="s")
    rpw = nrows // _NW
    nchunk = rpw // _R

    @functools.partial(
        pl.kernel,
        mesh=mesh,
        compiler_params=pltpu.CompilerParams(
            needs_layout_passes=False, use_tc_tiling_on_sc=True),
        out_type=jax.ShapeDtypeStruct((8 * nrows * 128,), jnp.float32),
        scratch_types=[
            pltpu.VMEM((_R, _LG), jnp.int32),
            pltpu.VMEM((_R, _LG), jnp.int32),
            pltpu.VMEM((_R, _LT), jnp.int32),
            pltpu.VMEM((_R, _LT), jnp.int32),
            pltpu.VMEM((_R * _W,), jnp.float32),
            pltpu.VMEM((_R * _W,), jnp.float32),
            pltpu.SemaphoreType.DMA,
            pltpu.SemaphoreType.DMA,
            pltpu.SemaphoreType.DMA,
            pltpu.SemaphoreType.DMA,
            pltpu.SemaphoreType.DMA,
            pltpu.SemaphoreType.DMA,
        ],
    )
    def hist_kernel(gid_hbm, tid_hbm, out_hbm, gid_v0, gid_v1, tid_v0, tid_v1,
                    hist_v0, hist_v1, sg0, sg1, st0, st1, so0, so1):
        gids = (gid_v0, gid_v1)
        tids = (tid_v0, tid_v1)
        hists = (hist_v0, hist_v1)
        sgs = (sg0, sg1)
        sts = (st0, st1)
        sos = (so0, so1)
        wid = lax.axis_index("s") * _NC + lax.axis_index("c")
        base = wid * rpw
        ones = jnp.ones((_LANES,), jnp.float32)
        tval = jnp.full((_LANES,), _TSCALE, jnp.float32)
        lane = lax.iota(jnp.int32, _LANES)
        tailmask = lane >= (16 - _LG % 16)
        zero16 = jnp.zeros((16,), jnp.float32)

        def start_id_load(c, b):
            row0 = base + c * _R
            pltpu.async_copy(gid_hbm.at[pl.ds(row0, _R)], gids[b], sgs[b])
            pltpu.async_copy(tid_hbm.at[pl.ds(row0, _R)], tids[b], sts[b])

        def wait_id(b):
            pltpu.make_async_copy(
                gid_hbm.at[pl.ds(0, _R)], gids[b], sgs[b]).wait()
            pltpu.make_async_copy(
                tid_hbm.at[pl.ds(0, _R)], tids[b], sts[b]).wait()

        start_id_load(0, 0)
        start_id_load(1, 1)

        def outer_body(o, carry):
            for b in range(2):
                c = o * 2 + b
                row0 = base + c * _R
                hist_b = hists[b]
                gid_b = gids[b]
                tid_b = tids[b]

                @pl.when(c >= 2)
                def _wait_out():
                    for s in range(8):
                        pltpu.make_async_copy(
                            hist_b.at[pl.ds(0, _R * 128)],
                            out_hbm.at[pl.ds(0, _R * 128)], sos[b]).wait()

                def zero_body(j, carry2):
                    for k in range(16):
                        hist_b[pl.ds(j * 256 + k * 16, 16)] = zero16
                    return carry2

                lax.fori_loop(0, (_R * _W) // 256, zero_body, 0)

                wait_id(b)

                def row_body(r, carry2):
                    rb = jnp.full((_LANES,), r * 128, jnp.int32)
                    chunks = (
                        [(cc * 16, gid_b, ones, None)
                         for cc in range(_LG // 16)]
                        + [(_LG - 16, gid_b, ones, tailmask)]
                        + [(cc * 16, tid_b, tval, None)
                           for cc in range(_LT // 16)]
                    )
                    for i in range(0, len(chunks), _GRP):
                        grp = chunks[i:i + _GRP]
                        vals = [ref[r, pl.ds(off, 16)]
                                for off, ref, _, _ in grp]
                        idxs = [((v & 896) << 4) + (v & 127) + rb
                                for v in vals]
                        for idx, (_, _, val, msk) in zip(idxs, grp):
                            if msk is None:
                                plsc.addupdate_scatter(hist_b, [idx], val)
                            else:
                                plsc.addupdate_scatter(
                                    hist_b, [idx], val, mask=msk)
                    return carry2

                lax.fori_loop(0, _R, row_body, 0)

                @pl.when(c + 2 < nchunk)
                def _prefetch():
                    start_id_load(c + 2, b)

                for s in range(8):
                    pltpu.async_copy(
                        hist_b.at[pl.ds(s * _R * 128, _R * 128)],
                        out_hbm.at[pl.ds((s * nrows + row0) * 128, _R * 128)],
                        sos[b])
            return carry

        lax.fori_loop(0, nchunk // 2, outer_body, 0)

        for b in range(2):
            for s in range(8):
                pltpu.make_async_copy(
                    hists[b].at[pl.ds(0, _R * 128)],
                    out_hbm.at[pl.ds(0, _R * 128)], sos[b]).wait()

    return hist_kernel(gid, tid)


def _tc_towers(counts2, Tg2, Tt2, Wg1, bg1, Wg2, bg2, Wg3, bg3,
               Wt1, bt1, Wt2, bt2, Wt3, bt3):
    nrows = counts2.shape[1]
    BB = min(2048, nrows)
    NCT = _W // 128

    def body(c_ref, tg_ref, tt_ref, wg1, bg1r, wg2, bg2r, wg3, bg3r,
             wt1, bt1r, wt2, bt2r, wt3, bt3r, out_ref, x_s):
        ct = pl.program_id(1)
        v = c_ref[0]
        tc_cnt = jnp.floor(v * (1.0 / 4096.0))
        gc_cnt = v - tc_cnt * 4096.0
        p = (jnp.dot(gc_cnt.astype(jnp.bfloat16), tg_ref[...],
                     preferred_element_type=jnp.float32)
             + jnp.dot(tc_cnt.astype(jnp.bfloat16), tt_ref[...],
                       preferred_element_type=jnp.float32))

        @pl.when(ct == 0)
        def _init():
            x_s[...] = p

        @pl.when(ct > 0)
        def _acc():
            x_s[...] = x_s[...] + p

        @pl.when(ct == NCT - 1)
        def _finish():
            x = x_s[...]
            g = (x[:, 0:_D]
                 / jnp.maximum(x[:, _D:_D + 1], 1.0)).astype(jnp.bfloat16)
            t = (x[:, 64:64 + _D]
                 / jnp.maximum(x[:, 64 + _D:64 + _D + 1],
                               1.0)).astype(jnp.bfloat16)
            hg = jnp.maximum(
                jnp.dot(g, wg1[...], preferred_element_type=jnp.float32)
                + bg1r[...], 0.0).astype(jnp.bfloat16)
            hg = jnp.maximum(
                jnp.dot(hg, wg2[...], preferred_element_type=jnp.float32)
                + bg2r[...], 0.0).astype(jnp.bfloat16)
            gv = (jnp.dot(hg, wg3[...], preferred_element_type=jnp.float32)
                  + bg3r[...])
            ht = jnp.maximum(
                jnp.dot(t, wt1[...], preferred_element_type=jnp.float32)
                + bt1r[...], 0.0).astype(jnp.bfloat16)
            ht = jnp.maximum(
                jnp.dot(ht, wt2[...], preferred_element_type=jnp.float32)
                + bt2r[...], 0.0).astype(jnp.bfloat16)
            tv = (jnp.dot(ht, wt3[...], preferred_element_type=jnp.float32)
                  + bt3r[...])
            out_ref[...] = jax.nn.sigmoid(jnp.sum(gv * tv, axis=1))

    full = lambda shape: pl.BlockSpec(shape, lambda i, ct: (0,) * len(shape))
    return pl.pallas_call(
        body,
        grid=(nrows // BB, NCT),
        in_specs=[
            pl.BlockSpec((1, BB, 128), lambda i, ct: (ct, i, 0)),
            pl.BlockSpec((128, 128), lambda i, ct: (ct, 0)),
            pl.BlockSpec((128, 128), lambda i, ct: (ct, 0)),
            full((_D, _H)), full((1, _H)),
            full((_H, _H)), full((1, _H)),
            full((_H, _OUT)), full((1, _OUT)),
            full((_D, _H)), full((1, _H)),
            full((_H, _H)), full((1, _H)),
            full((_H, _OUT)), full((1, _OUT)),
        ],
        out_specs=pl.BlockSpec((BB,), lambda i, ct: (i,)),
        out_shape=jax.ShapeDtypeStruct((nrows,), jnp.float32),
        scratch_shapes=[pltpu.VMEM((BB, 128), jnp.float32)],
    )(counts2, Tg2.astype(jnp.bfloat16), Tt2.astype(jnp.bfloat16),
      Wg1.astype(jnp.bfloat16), bg1.reshape(1, _H),
      Wg2.astype(jnp.bfloat16), bg2.reshape(1, _H),
      Wg3.astype(jnp.bfloat16), bg3.reshape(1, _OUT),
      Wt1.astype(jnp.bfloat16), bt1.reshape(1, _H),
      Wt2.astype(jnp.bfloat16), bt2.reshape(1, _H),
      Wt3.astype(jnp.bfloat16), bt3.reshape(1, _OUT))


def kernel(group_ids, tech_ids, Eg, Et, Wg1, bg1, Wg2, bg2, Wg3, bg3,
           Wt1, bt1, Wt2, bt2, Wt3, bt3):
    nsplit = 2
    qr = _B // nsplit
    counts_q = [
        _sc_histogram(group_ids[i * qr:(i + 1) * qr],
                      tech_ids[i * qr:(i + 1) * qr], qr)
        for i in range(nsplit)
    ]

    m = (jnp.arange(_V) != 0).astype(jnp.float32)[:, None]
    z = jnp.zeros((_V, 1), jnp.float32)
    tg = jnp.concatenate(
        [Eg * m, m] + [z] * (128 - _D - 1), axis=1)
    tt = jnp.concatenate(
        [z] * 64 + [Et * m, m] + [z] * (128 - 64 - _D - 1), axis=1)
    Tg = jnp.pad(tg, ((0, _W - _V), (0, 0)))
    Tt = jnp.pad(tt, ((0, _W - _V), (0, 0)))

    outs = [
        _tc_towers(c.reshape(8, c.shape[0] // (8 * 128), 128), Tg, Tt,
                   Wg1, bg1, Wg2, bg2, Wg3, bg3,
                   Wt1, bt1, Wt2, bt2, Wt3, bt3)
        for c in counts_q
    ]
    return jnp.concatenate(outs, axis=0)


# speedup vs baseline: 1.0496x; 1.0167x over previous
import functools

import jax
import jax.numpy as jnp
from jax import lax
from jax.experimental import pallas as pl
from jax.experimental.pallas import tpu as pltpu
from jax.experimental.pallas import tpu_sc as plsc

_B = 4096
_LG = 200
_LT = 400
_V = 1000
_D = 32
_H = 256
_OUT = 64
_W = 1024
_TSCALE = 4096.0

_NC = 2
_NS = 16
_LANES = 16
_NW = _NC * _NS
_RPW = _B // _NW
_R = 16
_NCHUNK = _RPW // _R
_GRP = 16


def _sc_histogram(gid, tid, nrows):
    mesh = plsc.VectorSubcoreMesh(core_axis_name="c", subcore_axis_name="s")
    rpw = nrows // _NW
    nchunk = rpw // _R

    @functools.partial(
        pl.kernel,
        mesh=mesh,
        compiler_params=pltpu.CompilerParams(needs_layout_passes=False),
        out_type=jax.ShapeDtypeStruct((8 * nrows * 128,), jnp.float32),
        scratch_types=[
            pltpu.VMEM((_R, _LG), jnp.int32),
            pltpu.VMEM((_R, _LG), jnp.int32),
            pltpu.VMEM((_R, _LT), jnp.int32),
            pltpu.VMEM((_R, _LT), jnp.int32),
            pltpu.VMEM((_R * _W,), jnp.float32),
            pltpu.VMEM((_R * _W,), jnp.float32),
            pltpu.SemaphoreType.DMA,
            pltpu.SemaphoreType.DMA,
            pltpu.SemaphoreType.DMA,
            pltpu.SemaphoreType.DMA,
            pltpu.SemaphoreType.DMA,
            pltpu.SemaphoreType.DMA,
        ],
    )
    def hist_kernel(gid_hbm, tid_hbm, out_hbm, gid_v0, gid_v1, tid_v0, tid_v1,
                    hist_v0, hist_v1, sg0, sg1, st0, st1, so0, so1):
        gids = (gid_v0, gid_v1)
        tids = (tid_v0, tid_v1)
        hists = (hist_v0, hist_v1)
        sgs = (sg0, sg1)
        sts = (st0, st1)
        sos = (so0, so1)
        wid = lax.axis_index("s") * _NC + lax.axis_index("c")
        base = wid * rpw
        ones = jnp.ones((_LANES,), jnp.float32)
        tval = jnp.full((_LANES,), _TSCALE, jnp.float32)
        lane = lax.iota(jnp.int32, _LANES)
        tailmask = lane >= (16 - _LG % 16)
        zero16 = jnp.zeros((16,), jnp.float32)

        def start_id_load(c, b):
            row0 = base + c * _R
            pltpu.async_copy(gid_hbm.at[pl.ds(row0, _R)], gids[b], sgs[b])
            pltpu.async_copy(tid_hbm.at[pl.ds(row0, _R)], tids[b], sts[b])

        def wait_id(b):
            pltpu.make_async_copy(
                gid_hbm.at[pl.ds(0, _R)], gids[b], sgs[b]).wait()
            pltpu.make_async_copy(
                tid_hbm.at[pl.ds(0, _R)], tids[b], sts[b]).wait()

        start_id_load(0, 0)
        start_id_load(1, 1)

        def outer_body(o, carry):
            for b in range(2):
                c = o * 2 + b
                row0 = base + c * _R
                hist_b = hists[b]
                gid_b = gids[b]
                tid_b = tids[b]

                @pl.when(c >= 2)
                def _wait_out():
                    for s in range(8):
                        pltpu.make_async_copy(
                            hist_b.at[pl.ds(0, _R * 128)],
                            out_hbm.at[pl.ds(0, _R * 128)], sos[b]).wait()

                def zero_body(j, carry2):
                    for k in range(16):
                        hist_b[pl.ds(j * 256 + k * 16, 16)] = zero16
                    return carry2

                lax.fori_loop(0, (_R * _W) // 256, zero_body, 0)

                wait_id(b)

                def row_body(r, carry2):
                    rb = jnp.full((_LANES,), r * 128, jnp.int32)
                    chunks = (
                        [(cc * 16, gid_b, ones, None)
                         for cc in range(_LG // 16)]
                        + [(_LG - 16, gid_b, ones, tailmask)]
                        + [(cc * 16, tid_b, tval, None)
                           for cc in range(_LT // 16)]
                    )
                    for i in range(0, len(chunks), _GRP):
                        grp = chunks[i:i + _GRP]
                        vals = [ref[r, pl.ds(off, 16)]
                                for off, ref, _, _ in grp]
                        idxs = [((v & 896) << 4) + (v & 127) + rb
                                for v in vals]
                        for idx, (_, _, val, msk) in zip(idxs, grp):
                            if msk is None:
                                plsc.addupdate_scatter(hist_b, [idx], val)
                            else:
                                plsc.addupdate_scatter(
                                    hist_b, [idx], val, mask=msk)
                    return carry2

                lax.fori_loop(0, _R, row_body, 0)

                @pl.when(c + 2 < nchunk)
                def _prefetch():
                    start_id_load(c + 2, b)

                for s in range(8):
                    pltpu.async_copy(
                        hist_b.at[pl.ds(s * _R * 128, _R * 128)],
                        out_hbm.at[pl.ds((s * nrows + row0) * 128, _R * 128)],
                        sos[b])
            return carry

        lax.fori_loop(0, nchunk // 2, outer_body, 0)

        for b in range(2):
            for s in range(8):
                pltpu.make_async_copy(
                    hists[b].at[pl.ds(0, _R * 128)],
                    out_hbm.at[pl.ds(0, _R * 128)], sos[b]).wait()

    return hist_kernel(gid, tid)


def _tc_towers(counts2, Tg2, Tt2, Wg1, bg1, Wg2, bg2, Wg3, bg3,
               Wt1, bt1, Wt2, bt2, Wt3, bt3):
    nrows = counts2.shape[1]
    BB = min(2048, nrows)
    NCT = _W // 128

    def body(c_ref, tg_ref, tt_ref, wg1, bg1r, wg2, bg2r, wg3, bg3r,
             wt1, bt1r, wt2, bt2r, wt3, bt3r, out_ref, x_s):
        ct = pl.program_id(1)
        v = c_ref[0]
        tc_cnt = jnp.floor(v * (1.0 / 4096.0))
        gc_cnt = v - tc_cnt * 4096.0
        p = (jnp.dot(gc_cnt.astype(jnp.bfloat16), tg_ref[...],
                     preferred_element_type=jnp.float32)
             + jnp.dot(tc_cnt.astype(jnp.bfloat16), tt_ref[...],
                       preferred_element_type=jnp.float32))

        @pl.when(ct == 0)
        def _init():
            x_s[...] = p

        @pl.when(ct > 0)
        def _acc():
            x_s[...] = x_s[...] + p

        @pl.when(ct == NCT - 1)
        def _finish():
            x = x_s[...]
            g = (x[:, 0:_D]
                 / jnp.maximum(x[:, _D:_D + 1], 1.0)).astype(jnp.bfloat16)
            t = (x[:, 64:64 + _D]
                 / jnp.maximum(x[:, 64 + _D:64 + _D + 1],
                               1.0)).astype(jnp.bfloat16)
            hg = jnp.maximum(
                jnp.dot(g, wg1[...], preferred_element_type=jnp.float32)
                + bg1r[...], 0.0).astype(jnp.bfloat16)
            hg = jnp.maximum(
                jnp.dot(hg, wg2[...], preferred_element_type=jnp.float32)
                + bg2r[...], 0.0).astype(jnp.bfloat16)
            gv = (jnp.dot(hg, wg3[...], preferred_element_type=jnp.float32)
                  + bg3r[...])
            ht = jnp.maximum(
                jnp.dot(t, wt1[...], preferred_element_type=jnp.float32)
                + bt1r[...], 0.0).astype(jnp.bfloat16)
            ht = jnp.maximum(
                jnp.dot(ht, wt2[...], preferred_element_type=jnp.float32)
                + bt2r[...], 0.0).astype(jnp.bfloat16)
            tv = (jnp.dot(ht, wt3[...], preferred_element_type=jnp.float32)
                  + bt3r[...])
            out_ref[...] = jax.nn.sigmoid(jnp.sum(gv * tv, axis=1))

    full = lambda shape: pl.BlockSpec(shape, lambda i, ct: (0,) * len(shape))
    return pl.pallas_call(
        body,
        grid=(nrows // BB, NCT),
        in_specs=[
            pl.BlockSpec((1, BB, 128), lambda i, ct: (ct, i, 0)),
            pl.BlockSpec((128, 128), lambda i, ct: (ct, 0)),
            pl.BlockSpec((128, 128), lambda i, ct: (ct, 0)),
            full((_D, _H)), full((1, _H)),
            full((_H, _H)), full((1, _H)),
            full((_H, _OUT)), full((1, _OUT)),
            full((_D, _H)), full((1, _H)),
            full((_H, _H)), full((1, _H)),
            full((_H, _OUT)), full((1, _OUT)),
        ],
        out_specs=pl.BlockSpec((BB,), lambda i, ct: (i,)),
        out_shape=jax.ShapeDtypeStruct((nrows,), jnp.float32),
        scratch_shapes=[pltpu.VMEM((BB, 128), jnp.float32)],
    )(counts2, Tg2.astype(jnp.bfloat16), Tt2.astype(jnp.bfloat16),
      Wg1.astype(jnp.bfloat16), bg1.reshape(1, _H),
      Wg2.astype(jnp.bfloat16), bg2.reshape(1, _H),
      Wg3.astype(jnp.bfloat16), bg3.reshape(1, _OUT),
      Wt1.astype(jnp.bfloat16), bt1.reshape(1, _H),
      Wt2.astype(jnp.bfloat16), bt2.reshape(1, _H),
      Wt3.astype(jnp.bfloat16), bt3.reshape(1, _OUT))


def kernel(group_ids, tech_ids, Eg, Et, Wg1, bg1, Wg2, bg2, Wg3, bg3,
           Wt1, bt1, Wt2, bt2, Wt3, bt3):
    nsplit = 2
    qr = _B // nsplit
    counts_q = [
        _sc_histogram(group_ids[i * qr:(i + 1) * qr],
                      tech_ids[i * qr:(i + 1) * qr], qr)
        for i in range(nsplit)
    ]

    m = (jnp.arange(_V) != 0).astype(jnp.float32)[:, None]
    z = jnp.zeros((_V, 1), jnp.float32)
    tg = jnp.concatenate(
        [Eg * m, m] + [z] * (128 - _D - 1), axis=1)
    tt = jnp.concatenate(
        [z] * 64 + [Et * m, m] + [z] * (128 - 64 - _D - 1), axis=1)
    Tg = jnp.pad(tg, ((0, _W - _V), (0, 0)))
    Tt = jnp.pad(tt, ((0, _W - _V), (0, 0)))

    outs = [
        _tc_towers(c.reshape(8, c.shape[0] // (8 * 128), 128), Tg, Tt,
                   Wg1, bg1, Wg2, bg2, Wg3, bg3,
                   Wt1, bt1, Wt2, bt2, Wt3, bt3)
        for c in counts_q
    ]
    return jnp.concatenate(outs, axis=0)


# speedup vs baseline: 1.0514x; 1.0017x over previous
import functools

import jax
import jax.numpy as jnp
from jax import lax
from jax.experimental import pallas as pl
from jax.experimental.pallas import tpu as pltpu
from jax.experimental.pallas import tpu_sc as plsc

_B = 4096
_LG = 200
_LT = 400
_V = 1000
_D = 32
_H = 256
_OUT = 64
_W = 1024
_TSCALE = 4096.0

_NC = 2
_NS = 16
_LANES = 16
_NW = _NC * _NS
_RPW = _B // _NW
_R = 16
_NCHUNK = _RPW // _R
_GRP = 19


def _sc_histogram(gid, tid, nrows):
    mesh = plsc.VectorSubcoreMesh(core_axis_name="c", subcore_axis_name="s")
    rpw = nrows // _NW
    nchunk = rpw // _R

    @functools.partial(
        pl.kernel,
        mesh=mesh,
        compiler_params=pltpu.CompilerParams(needs_layout_passes=False),
        out_type=jax.ShapeDtypeStruct((8 * nrows * 128,), jnp.float32),
        scratch_types=[
            pltpu.VMEM((_R, _LG), jnp.int32),
            pltpu.VMEM((_R, _LG), jnp.int32),
            pltpu.VMEM((_R, _LT), jnp.int32),
            pltpu.VMEM((_R, _LT), jnp.int32),
            pltpu.VMEM((_R * _W,), jnp.float32),
            pltpu.VMEM((_R * _W,), jnp.float32),
            pltpu.SemaphoreType.DMA,
            pltpu.SemaphoreType.DMA,
            pltpu.SemaphoreType.DMA,
            pltpu.SemaphoreType.DMA,
            pltpu.SemaphoreType.DMA,
            pltpu.SemaphoreType.DMA,
        ],
    )
    def hist_kernel(gid_hbm, tid_hbm, out_hbm, gid_v0, gid_v1, tid_v0, tid_v1,
                    hist_v0, hist_v1, sg0, sg1, st0, st1, so0, so1):
        gids = (gid_v0, gid_v1)
        tids = (tid_v0, tid_v1)
        hists = (hist_v0, hist_v1)
        sgs = (sg0, sg1)
        sts = (st0, st1)
        sos = (so0, so1)
        wid = lax.axis_index("s") * _NC + lax.axis_index("c")
        base = wid * rpw
        ones = jnp.ones((_LANES,), jnp.float32)
        tval = jnp.full((_LANES,), _TSCALE, jnp.float32)
        lane = lax.iota(jnp.int32, _LANES)
        tailmask = lane >= (16 - _LG % 16)
        zero16 = jnp.zeros((16,), jnp.float32)

        def start_id_load(c, b):
            row0 = base + c * _R
            pltpu.async_copy(gid_hbm.at[pl.ds(row0, _R)], gids[b], sgs[b])
            pltpu.async_copy(tid_hbm.at[pl.ds(row0, _R)], tids[b], sts[b])

        def wait_id(b):
            pltpu.make_async_copy(
                gid_hbm.at[pl.ds(0, _R)], gids[b], sgs[b]).wait()
            pltpu.make_async_copy(
                tid_hbm.at[pl.ds(0, _R)], tids[b], sts[b]).wait()

        start_id_load(0, 0)
        start_id_load(1, 1)

        def outer_body(o, carry):
            for b in range(2):
                c = o * 2 + b
                row0 = base + c * _R
                hist_b = hists[b]
                gid_b = gids[b]
                tid_b = tids[b]

                @pl.when(c >= 2)
                def _wait_out():
                    for s in range(8):
                        pltpu.make_async_copy(
                            hist_b.at[pl.ds(0, _R * 128)],
                            out_hbm.at[pl.ds(0, _R * 128)], sos[b]).wait()

                def zero_body(j, carry2):
                    for k in range(16):
                        hist_b[pl.ds(j * 256 + k * 16, 16)] = zero16
                    return carry2

                lax.fori_loop(0, (_R * _W) // 256, zero_body, 0)

                wait_id(b)

                def row_body(r, carry2):
                    rb = jnp.full((_LANES,), r * 128, jnp.int32)
                    chunks = (
                        [(cc * 16, gid_b, ones, None)
                         for cc in range(_LG // 16)]
                        + [(_LG - 16, gid_b, ones, tailmask)]
                        + [(cc * 16, tid_b, tval, None)
                           for cc in range(_LT // 16)]
                    )
                    for i in range(0, len(chunks), _GRP):
                        grp = chunks[i:i + _GRP]
                        vals = [ref[r, pl.ds(off, 16)]
                                for off, ref, _, _ in grp]
                        idxs = [((v & 896) << 4) + (v & 127) + rb
                                for v in vals]
                        for idx, (_, _, val, msk) in zip(idxs, grp):
                            if msk is None:
                                plsc.addupdate_scatter(hist_b, [idx], val)
                            else:
                                plsc.addupdate_scatter(
                                    hist_b, [idx], val, mask=msk)
                    return carry2

                lax.fori_loop(0, _R, row_body, 0)

                @pl.when(c + 2 < nchunk)
                def _prefetch():
                    start_id_load(c + 2, b)

                for s in range(8):
                    pltpu.async_copy(
                        hist_b.at[pl.ds(s * _R * 128, _R * 128)],
                        out_hbm.at[pl.ds((s * nrows + row0) * 128, _R * 128)],
                        sos[b])
            return carry

        lax.fori_loop(0, nchunk // 2, outer_body, 0)

        for b in range(2):
            for s in range(8):
                pltpu.make_async_copy(
                    hists[b].at[pl.ds(0, _R * 128)],
                    out_hbm.at[pl.ds(0, _R * 128)], sos[b]).wait()

    return hist_kernel(gid, tid)


def _tc_towers(counts2, Tg2, Tt2, Wg1, bg1, Wg2, bg2, Wg3, bg3,
               Wt1, bt1, Wt2, bt2, Wt3, bt3):
    nrows = counts2.shape[1]
    BB = min(2048, nrows)
    NCT = _W // 128

    def body(c_ref, tg_ref, tt_ref, wg1, bg1r, wg2, bg2r, wg3, bg3r,
             wt1, bt1r, wt2, bt2r, wt3, bt3r, out_ref, x_s):
        ct = pl.program_id(1)
        v = c_ref[0]
        tc_cnt = jnp.floor(v * (1.0 / 4096.0))
        gc_cnt = v - tc_cnt * 4096.0
        p = (jnp.dot(gc_cnt.astype(jnp.bfloat16), tg_ref[...],
                     preferred_element_type=jnp.float32)
             + jnp.dot(tc_cnt.astype(jnp.bfloat16), tt_ref[...],
                       preferred_element_type=jnp.float32))

        @pl.when(ct == 0)
        def _init():
            x_s[...] = p

        @pl.when(ct > 0)
        def _acc():
            x_s[...] = x_s[...] + p

        @pl.when(ct == NCT - 1)
        def _finish():
            x = x_s[...]
            g = (x[:, 0:_D]
                 / jnp.maximum(x[:, _D:_D + 1], 1.0)).astype(jnp.bfloat16)
            t = (x[:, 64:64 + _D]
                 / jnp.maximum(x[:, 64 + _D:64 + _D + 1],
                               1.0)).astype(jnp.bfloat16)
            hg = jnp.maximum(
                jnp.dot(g, wg1[...], preferred_element_type=jnp.float32)
                + bg1r[...], 0.0).astype(jnp.bfloat16)
            hg = jnp.maximum(
                jnp.dot(hg, wg2[...], preferred_element_type=jnp.float32)
                + bg2r[...], 0.0).astype(jnp.bfloat16)
            gv = (jnp.dot(hg, wg3[...], preferred_element_type=jnp.float32)
                  + bg3r[...])
            ht = jnp.maximum(
                jnp.dot(t, wt1[...], preferred_element_type=jnp.float32)
                + bt1r[...], 0.0).astype(jnp.bfloat16)
            ht = jnp.maximum(
                jnp.dot(ht, wt2[...], preferred_element_type=jnp.float32)
                + bt2r[...], 0.0).astype(jnp.bfloat16)
            tv = (jnp.dot(ht, wt3[...], preferred_element_type=jnp.float32)
                  + bt3r[...])
            out_ref[...] = jax.nn.sigmoid(jnp.sum(gv * tv, axis=1))

    full = lambda shape: pl.BlockSpec(shape, lambda i, ct: (0,) * len(shape))
    return pl.pallas_call(
        body,
        grid=(nrows // BB, NCT),
        in_specs=[
            pl.BlockSpec((1, BB, 128), lambda i, ct: (ct, i, 0)),
            pl.BlockSpec((128, 128), lambda i, ct: (ct, 0)),
            pl.BlockSpec((128, 128), lambda i, ct: (ct, 0)),
            full((_D, _H)), full((1, _H)),
            full((_H, _H)), full((1, _H)),
            full((_H, _OUT)), full((1, _OUT)),
            full((_D, _H)), full((1, _H)),
            full((_H, _H)), full((1, _H)),
            full((_H, _OUT)), full((1, _OUT)),
        ],
        out_specs=pl.BlockSpec((BB,), lambda i, ct: (i,)),
        out_shape=jax.ShapeDtypeStruct((nrows,), jnp.float32),
        scratch_shapes=[pltpu.VMEM((BB, 128), jnp.float32)],
    )(counts2, Tg2.astype(jnp.bfloat16), Tt2.astype(jnp.bfloat16),
      Wg1.astype(jnp.bfloat16), bg1.reshape(1, _H),
      Wg2.astype(jnp.bfloat16), bg2.reshape(1, _H),
      Wg3.astype(jnp.bfloat16), bg3.reshape(1, _OUT),
      Wt1.astype(jnp.bfloat16), bt1.reshape(1, _H),
      Wt2.astype(jnp.bfloat16), bt2.reshape(1, _H),
      Wt3.astype(jnp.bfloat16), bt3.reshape(1, _OUT))


def kernel(group_ids, tech_ids, Eg, Et, Wg1, bg1, Wg2, bg2, Wg3, bg3,
           Wt1, bt1, Wt2, bt2, Wt3, bt3):
    nsplit = 2
    qr = _B // nsplit
    counts_q = [
        _sc_histogram(group_ids[i * qr:(i + 1) * qr],
                      tech_ids[i * qr:(i + 1) * qr], qr)
        for i in range(nsplit)
    ]

    m = (jnp.arange(_V) != 0).astype(jnp.float32)[:, None]
    z = jnp.zeros((_V, 1), jnp.float32)
    tg = jnp.concatenate(
        [Eg * m, m] + [z] * (128 - _D - 1), axis=1)
    tt = jnp.concatenate(
        [z] * 64 + [Et * m, m] + [z] * (128 - 64 - _D - 1), axis=1)
    Tg = jnp.pad(tg, ((0, _W - _V), (0, 0)))
    Tt = jnp.pad(tt, ((0, _W - _V), (0, 0)))

    outs = [
        _tc_towers(c.reshape(8, c.shape[0] // (8 * 128), 128), Tg, Tt,
                   Wg1, bg1, Wg2, bg2, Wg3, bg3,
                   Wt1, bt1, Wt2, bt2, Wt3, bt3)
        for c in counts_q
    ]
    return jnp.concatenate(outs, axis=0)
